# Initial kernel scaffold; baseline (speedup 1.0000x reference)
import jax
import jax.numpy as jnp
from jax.experimental import pallas as pl


def kernel(x, W, bias):
    raise NotImplementedError("write your pallas kernel here")


# speedup vs baseline: 1.8087x; 1.8087x over previous
import functools

import jax
import jax.numpy as jnp
from jax import lax
from jax.experimental import pallas as pl
from jax.experimental.pallas import tpu as pltpu
from jax.experimental.pallas import tpu_sc as plsc

NUM_FIELDS = 26
EMBED_DIM = 416
BLK = 16
BATCH = 4096
FIELD_SIZE = 2000

NC, NS = 2, 16
NW = NC * NS
CB = BATCH // NW
G = 4
ROWS = G * NUM_FIELDS
NCHUNK = CB // G

_MESH = plsc.VectorSubcoreMesh(core_axis_name="c", subcore_axis_name="s")


@functools.partial(
    pl.kernel,
    out_type=jax.ShapeDtypeStruct((BATCH,), jnp.float32),
    mesh=_MESH,
    scratch_types=[
        pltpu.VMEM((NCHUNK, ROWS), jnp.int32),
        pltpu.VMEM((2, ROWS, EMBED_DIM), jnp.float32),
        pltpu.VMEM((CB,), jnp.float32),
        pltpu.SemaphoreType.DMA,
        pltpu.SemaphoreType.DMA,
    ],
    compiler_params=pltpu.CompilerParams(
        needs_layout_passes=False, use_tc_tiling_on_sc=False
    ),
)
def _fwlm_sc(idx_hbm, w_hbm, out_hbm, idx_v, rows_v, out_v, sem0, sem1):
    wid = lax.axis_index("s") * NC + lax.axis_index("c")
    sems = (sem0, sem1)
    lanes = lax.iota(jnp.int32, BLK)

    for i in range(CB // BLK):
        out_v[pl.ds(i * BLK, BLK)] = jnp.zeros((BLK,), jnp.float32)

    pltpu.sync_copy(idx_hbm.at[wid], idx_v)

    def fire(c, b):
        pltpu.async_copy(w_hbm.at[idx_v.at[c]], rows_v.at[b], sems[b])

    def wait(c, b):
        pltpu.make_async_copy(w_hbm.at[idx_v.at[c]], rows_v.at[b], sems[b]).wait()

    def compute_chunk(c, b):
        def elem_body(e, carry):
            base = e * NUM_FIELDS
            acc = jnp.zeros((BLK,), jnp.float32)
            for j in range(NUM_FIELDS):
                col = pl.ds(BLK * j, BLK)
                s = rows_v[b, base, col]
                ej = s
                for f in range(1, NUM_FIELDS):
                    v = rows_v[b, base + f, col]
                    if f == j:
                        ej = v
                    s = s + v
                acc = acc + ej * (s - ej)
            s_val = jnp.sum(acc)
            pos = c * G + e
            grp = (pos // BLK) * BLK
            cur = out_v[pl.ds(grp, BLK)]
            out_v[pl.ds(grp, BLK)] = cur + jnp.where(
                lanes == pos % BLK, s_val, 0.0
            )
            return carry

        lax.fori_loop(0, G, elem_body, 0)

    fire(0, 0)
    fire(1, 1)

    def outer(g2, carry):
        for b in range(2):
            c = 2 * g2 + b
            wait(c, b)
            compute_chunk(c, b)
            fire(c + 2, b)
        return carry

    lax.fori_loop(0, NCHUNK // 2 - 1, outer, 0)

    for b in range(2):
        c = NCHUNK - 2 + b
        wait(c, b)
        compute_chunk(c, b)

    pltpu.sync_copy(out_v, out_hbm.at[pl.ds(wid * CB, CB)])


def kernel(x, W, bias):
    offs = (jnp.arange(NUM_FIELDS, dtype=jnp.int32) * FIELD_SIZE)[None, :]
    idx = (x.astype(jnp.int32) + offs).reshape(NW, NCHUNK, ROWS)
    out = _fwlm_sc(idx, W)
    return out + bias[0]


# speedup vs baseline: 1.8110x; 1.0013x over previous
import functools

import jax
import jax.numpy as jnp
from jax import lax
from jax.experimental import pallas as pl
from jax.experimental.pallas import tpu as pltpu
from jax.experimental.pallas import tpu_sc as plsc

NUM_FIELDS = 26
EMBED_DIM = 416
BLK = 16
BATCH = 4096
FIELD_SIZE = 2000

NC, NS = 2, 16
NW = NC * NS
CB = BATCH // NW
G = 4
ROWS = G * NUM_FIELDS
NCHUNK = CB // G

_MESH = plsc.VectorSubcoreMesh(core_axis_name="c", subcore_axis_name="s")


@functools.partial(
    pl.kernel,
    out_type=jax.ShapeDtypeStruct((BATCH,), jnp.float32),
    mesh=_MESH,
    scratch_types=[
        pltpu.VMEM((NCHUNK, ROWS), jnp.int32),
        pltpu.VMEM((2, ROWS, EMBED_DIM), jnp.float32),
        pltpu.VMEM((CB,), jnp.float32),
        pltpu.SemaphoreType.DMA,
        pltpu.SemaphoreType.DMA,
    ],
    compiler_params=pltpu.CompilerParams(
        needs_layout_passes=False, use_tc_tiling_on_sc=False
    ),
)
def _fwlm_sc(idx_hbm, w_hbm, out_hbm, idx_v, rows_v, out_v, sem0, sem1):
    wid = lax.axis_index("s") * NC + lax.axis_index("c")
    sems = (sem0, sem1)
    lanes = lax.iota(jnp.int32, BLK)

    for i in range(CB // BLK):
        out_v[pl.ds(i * BLK, BLK)] = jnp.zeros((BLK,), jnp.float32)

    pltpu.sync_copy(idx_hbm.at[wid], idx_v)

    def fire(c, b):
        pltpu.async_copy(w_hbm.at[idx_v.at[c]], rows_v.at[b], sems[b])

    def wait(c, b):
        pltpu.make_async_copy(w_hbm.at[idx_v.at[c]], rows_v.at[b], sems[b]).wait()

    def compute_chunk(c, b):
        def elem_body(e, carry):
            base = e * NUM_FIELDS
            acc = jnp.zeros((BLK,), jnp.float32)
            for j in range(NUM_FIELDS):
                col = pl.ds(BLK * j, BLK)
                s = rows_v[b, base, col]
                ej = s
                for f in range(1, NUM_FIELDS):
                    v = rows_v[b, base + f, col]
                    if f == j:
                        ej = v
                    s = s + v
                acc = acc + ej * (s - ej)
            s_val = jnp.sum(acc)
            pos = c * G + e
            grp = (pos // BLK) * BLK
            cur = out_v[pl.ds(grp, BLK)]
            out_v[pl.ds(grp, BLK)] = cur + jnp.where(
                lanes == pos % BLK, s_val, 0.0
            )
            return carry

        lax.fori_loop(0, G, elem_body, 0)

    fire(0, 0)
    fire(1, 1)

    def outer(g2, carry):
        for b in range(2):
            c = 2 * g2 + b
            wait(c, b)
            compute_chunk(c, b)
            fire(c + 2, b)
        return carry

    lax.fori_loop(0, NCHUNK // 2 - 1, outer, 0)

    for b in range(2):
        c = NCHUNK - 2 + b
        wait(c, b)
        compute_chunk(c, b)

    pltpu.sync_copy(out_v, out_hbm.at[pl.ds(wid * CB, CB)])


def kernel(x, W, bias):
    offs = (jnp.arange(NUM_FIELDS, dtype=jnp.int32) * FIELD_SIZE)[None, :]
    idx = (x.astype(jnp.int32) + offs).reshape(NW, NCHUNK, ROWS)
    w_lin = lax.optimization_barrier(W.reshape(-1)).reshape(W.shape)
    out = _fwlm_sc(idx, w_lin)
    return out + bias[0]


# speedup vs baseline: 1.9713x; 1.0885x over previous
import functools

import jax
import jax.numpy as jnp
from jax import lax
from jax.experimental import pallas as pl
from jax.experimental.pallas import tpu as pltpu
from jax.experimental.pallas import tpu_sc as plsc

NUM_FIELDS = 26
EMBED_DIM = 416
BLK = 16
BATCH = 4096
FIELD_SIZE = 2000

EMBED_PAD = 512
NC, NS = 2, 16
NW = NC * NS
CB = BATCH // NW
G = 4
ROWS = G * NUM_FIELDS
NCHUNK = CB // G

_MESH = plsc.VectorSubcoreMesh(core_axis_name="c", subcore_axis_name="s")


@functools.partial(
    pl.kernel,
    out_type=jax.ShapeDtypeStruct((BATCH,), jnp.float32),
    mesh=_MESH,
    scratch_types=[
        pltpu.VMEM((NCHUNK, ROWS), jnp.int32),
        pltpu.VMEM((2, ROWS, EMBED_PAD), jnp.float32),
        pltpu.VMEM((CB,), jnp.float32),
        pltpu.SemaphoreType.DMA,
        pltpu.SemaphoreType.DMA,
    ],
    compiler_params=pltpu.CompilerParams(
        needs_layout_passes=False, use_tc_tiling_on_sc=True
    ),
)
def _fwlm_sc(idx_hbm, w_hbm, out_hbm, idx_v, rows_v, out_v, sem0, sem1):
    wid = lax.axis_index("s") * NC + lax.axis_index("c")
    sems = (sem0, sem1)
    lanes = lax.iota(jnp.int32, BLK)

    for i in range(CB // BLK):
        out_v[pl.ds(i * BLK, BLK)] = jnp.zeros((BLK,), jnp.float32)

    pltpu.sync_copy(idx_hbm.at[wid], idx_v)

    def fire(c, b):
        pltpu.async_copy(w_hbm.at[idx_v.at[c]], rows_v.at[b], sems[b])

    def wait(c, b):
        pltpu.make_async_copy(w_hbm.at[idx_v.at[c]], rows_v.at[b], sems[b]).wait()

    def compute_chunk(c, b):
        def elem_body(e, carry):
            base = e * NUM_FIELDS
            acc = jnp.zeros((BLK,), jnp.float32)
            for j in range(NUM_FIELDS):
                col = pl.ds(BLK * j, BLK)
                s = rows_v[b, base, col]
                ej = s
                for f in range(1, NUM_FIELDS):
                    v = rows_v[b, base + f, col]
                    if f == j:
                        ej = v
                    s = s + v
                acc = acc + ej * (s - ej)
            s_val = jnp.sum(acc)
            pos = c * G + e
            grp = (pos // BLK) * BLK
            cur = out_v[pl.ds(grp, BLK)]
            out_v[pl.ds(grp, BLK)] = cur + jnp.where(
                lanes == pos % BLK, s_val, 0.0
            )
            return carry

        lax.fori_loop(0, G, elem_body, 0)

    fire(0, 0)
    fire(1, 1)

    def outer(g2, carry):
        for b in range(2):
            c = 2 * g2 + b
            wait(c, b)
            compute_chunk(c, b)
            fire(c + 2, b)
        return carry

    lax.fori_loop(0, NCHUNK // 2 - 1, outer, 0)

    for b in range(2):
        c = NCHUNK - 2 + b
        wait(c, b)
        compute_chunk(c, b)

    pltpu.sync_copy(out_v, out_hbm.at[pl.ds(wid * CB, CB)])


def kernel(x, W, bias):
    offs = (jnp.arange(NUM_FIELDS, dtype=jnp.int32) * FIELD_SIZE)[None, :]
    idx = (x.astype(jnp.int32) + offs).reshape(NW, NCHUNK, ROWS)
    w_pad = jnp.pad(W, ((0, 0), (0, EMBED_PAD - EMBED_DIM)))
    out = _fwlm_sc(idx, w_pad)
    return out + bias[0]


# speedup vs baseline: 4.5636x; 2.3150x over previous
import functools

import jax
import jax.numpy as jnp
from jax import lax
from jax.experimental import pallas as pl
from jax.experimental.pallas import tpu as pltpu
from jax.experimental.pallas import tpu_sc as plsc

NUM_FIELDS = 26
EMBED_DIM = 416
BLK = 16
BATCH = 4096
FIELD_SIZE = 2000

NC, NS = 2, 16
NW = NC * NS
CB = BATCH // NW
G = 4
ROWS = G * NUM_FIELDS
NCHUNK = CB // G

PANEL = 128
PANEL_OFFS = (0, 128, 256, 384)
NPANEL = len(PANEL_OFFS)

_MESH = plsc.VectorSubcoreMesh(core_axis_name="c", subcore_axis_name="s")


def _panel_of(col):
    for p in reversed(range(NPANEL)):
        if col >= PANEL_OFFS[p] and col + BLK <= PANEL_OFFS[p] + PANEL:
            return p, col - PANEL_OFFS[p]
    raise AssertionError(col)


@functools.partial(
    pl.kernel,
    out_type=jax.ShapeDtypeStruct((BATCH,), jnp.float32),
    mesh=_MESH,
    scratch_types=[
        pltpu.VMEM((NCHUNK, ROWS), jnp.int32),
        pltpu.VMEM((2, ROWS, NPANEL * PANEL), jnp.float32),
        pltpu.VMEM((CB,), jnp.float32),
        pltpu.SemaphoreType.DMA,
        pltpu.SemaphoreType.DMA,
    ],
    compiler_params=pltpu.CompilerParams(
        needs_layout_passes=False, use_tc_tiling_on_sc=True
    ),
)
def _fwlm_sc(idx_hbm, w_hbm, wtail_hbm, out_hbm, idx_v, rows_v, out_v, sem0, sem1):
    wid = lax.axis_index("s") * NC + lax.axis_index("c")
    sems = (sem0, sem1)
    lanes = lax.iota(jnp.int32, BLK)

    for i in range(CB // BLK):
        out_v[pl.ds(i * BLK, BLK)] = jnp.zeros((BLK,), jnp.float32)

    pltpu.sync_copy(idx_hbm.at[wid], idx_v)

    def _src(c, p):
        if p < NPANEL - 1:
            return w_hbm.at[idx_v.at[c], pl.ds(PANEL_OFFS[p], PANEL)]
        return wtail_hbm.at[idx_v.at[c]]

    def _dst(b, p):
        return rows_v.at[b, :, pl.ds(p * PANEL, PANEL)]

    def fire(c, b):
        for p in range(NPANEL):
            pltpu.async_copy(_src(c, p), _dst(b, p), sems[b])

    def wait(c, b):
        for p in range(NPANEL):
            pltpu.make_async_copy(_src(c, p), _dst(b, p), sems[b]).wait()

    def compute_chunk(c, b):
        def elem_body(e, carry):
            base = e * NUM_FIELDS
            acc = jnp.zeros((BLK,), jnp.float32)
            for j in range(NUM_FIELDS):
                p, off = _panel_of(BLK * j)
                col = pl.ds(p * PANEL + off, BLK)
                s = rows_v[b, base, col]
                ej = s
                for f in range(1, NUM_FIELDS):
                    v = rows_v[b, base + f, col]
                    if f == j:
                        ej = v
                    s = s + v
                acc = acc + ej * (s - ej)
            s_val = jnp.sum(acc)
            pos = c * G + e
            grp = (pos // BLK) * BLK
            cur = out_v[pl.ds(grp, BLK)]
            out_v[pl.ds(grp, BLK)] = cur + jnp.where(
                lanes == pos % BLK, s_val, 0.0
            )
            return carry

        lax.fori_loop(0, G, elem_body, 0)

    fire(0, 0)
    fire(1, 1)

    def outer(g2, carry):
        for b in range(2):
            c = 2 * g2 + b
            wait(c, b)
            compute_chunk(c, b)
            fire(c + 2, b)
        return carry

    lax.fori_loop(0, NCHUNK // 2 - 1, outer, 0)

    for b in range(2):
        c = NCHUNK - 2 + b
        wait(c, b)
        compute_chunk(c, b)

    pltpu.sync_copy(out_v, out_hbm.at[pl.ds(wid * CB, CB)])


def kernel(x, W, bias):
    offs = (jnp.arange(NUM_FIELDS, dtype=jnp.int32) * FIELD_SIZE)[None, :]
    idx = (x.astype(jnp.int32) + offs).reshape(NW, NCHUNK, ROWS)
    w_tail = jnp.pad(
        lax.slice(W, (0, 384), (W.shape[0], EMBED_DIM)),
        ((0, 0), (0, PANEL - (EMBED_DIM - 384))),
    )
    out = _fwlm_sc(idx, W, w_tail)
    return out + bias[0]


# speedup vs baseline: 5.0936x; 1.1161x over previous
import functools

import jax
import jax.numpy as jnp
from jax import lax
from jax.experimental import pallas as pl
from jax.experimental.pallas import tpu as pltpu
from jax.experimental.pallas import tpu_sc as plsc

NUM_FIELDS = 26
EMBED_DIM = 416
EMBED_PAD = 512
BLK = 16
BATCH = 4096
FIELD_SIZE = 2000
N_ROWS = NUM_FIELDS * FIELD_SIZE

NC, NS = 2, 16
NW = NC * NS
CB = BATCH // NW
G = 4
ROWS = G * NUM_FIELDS
NCHUNK = CB // G

TBLK = 512
TGRID = -(-N_ROWS // TBLK)

_MESH = plsc.VectorSubcoreMesh(core_axis_name="c", subcore_axis_name="s")


def _transpose_body(v_ref, o_ref):
    o_ref[:, :EMBED_DIM] = jnp.transpose(v_ref[...])
    o_ref[:, EMBED_DIM:] = jnp.zeros((TBLK, EMBED_PAD - EMBED_DIM), jnp.float32)


_transpose_tc = pl.pallas_call(
    _transpose_body,
    grid=(TGRID,),
    in_specs=[pl.BlockSpec((EMBED_DIM, TBLK), lambda i: (0, i))],
    out_specs=pl.BlockSpec((TBLK, EMBED_PAD), lambda i: (i, 0)),
    out_shape=jax.ShapeDtypeStruct((N_ROWS, EMBED_PAD), jnp.float32),
)


@functools.partial(
    pl.kernel,
    out_type=jax.ShapeDtypeStruct((BATCH,), jnp.float32),
    mesh=_MESH,
    scratch_types=[
        pltpu.VMEM((NCHUNK, ROWS), jnp.int32),
        pltpu.VMEM((2, ROWS, EMBED_PAD), jnp.float32),
        pltpu.VMEM((CB,), jnp.float32),
        pltpu.SemaphoreType.DMA,
        pltpu.SemaphoreType.DMA,
    ],
    compiler_params=pltpu.CompilerParams(
        needs_layout_passes=False, use_tc_tiling_on_sc=True
    ),
)
def _fwlm_sc(idx_hbm, w_hbm, out_hbm, idx_v, rows_v, out_v, sem0, sem1):
    wid = lax.axis_index("s") * NC + lax.axis_index("c")
    sems = (sem0, sem1)
    lanes = lax.iota(jnp.int32, BLK)

    for i in range(CB // BLK):
        out_v[pl.ds(i * BLK, BLK)] = jnp.zeros((BLK,), jnp.float32)

    pltpu.sync_copy(idx_hbm.at[wid], idx_v)

    def fire(c, b):
        pltpu.async_copy(w_hbm.at[idx_v.at[c]], rows_v.at[b], sems[b])

    def wait(c, b):
        pltpu.make_async_copy(w_hbm.at[idx_v.at[c]], rows_v.at[b], sems[b]).wait()

    def compute_chunk(c, b):
        def elem_body(e, carry):
            base = e * NUM_FIELDS
            acc = jnp.zeros((BLK,), jnp.float32)
            for j in range(NUM_FIELDS):
                col = pl.ds(BLK * j, BLK)
                s = rows_v[b, base, col]
                ej = s
                for f in range(1, NUM_FIELDS):
                    v = rows_v[b, base + f, col]
                    if f == j:
                        ej = v
                    s = s + v
                acc = acc + ej * (s - ej)
            s_val = jnp.sum(acc)
            pos = c * G + e
            grp = (pos // BLK) * BLK
            cur = out_v[pl.ds(grp, BLK)]
            out_v[pl.ds(grp, BLK)] = cur + jnp.where(
                lanes == pos % BLK, s_val, 0.0
            )
            return carry

        lax.fori_loop(0, G, elem_body, 0)

    fire(0, 0)
    fire(1, 1)

    def outer(g2, carry):
        for b in range(2):
            c = 2 * g2 + b
            wait(c, b)
            compute_chunk(c, b)
            fire(c + 2, b)
        return carry

    lax.fori_loop(0, NCHUNK // 2 - 1, outer, 0)

    for b in range(2):
        c = NCHUNK - 2 + b
        wait(c, b)
        compute_chunk(c, b)

    pltpu.sync_copy(out_v, out_hbm.at[pl.ds(wid * CB, CB)])


def kernel(x, W, bias):
    offs = (jnp.arange(NUM_FIELDS, dtype=jnp.int32) * FIELD_SIZE)[None, :]
    idx = (x.astype(jnp.int32) + offs).reshape(NW, NCHUNK, ROWS)
    w_pad = _transpose_tc(W.T)
    out = _fwlm_sc(idx, w_pad)
    return out + bias[0]
